# baseline (speedup 1.0000x reference)
import functools

import jax
import jax.numpy as jnp
from jax import lax
from jax.experimental import pallas as pl
from jax.experimental.pallas import tpu as pltpu
from jax.experimental.pallas import tpu_sc as plsc

D = 1024
F = 4096
E = 8
K = 2
N = 2048
C = 320
CP = 352
SP = E * CP
TRASH = 4096

NC, NS = 2, 16
NW = NC * NS
SPW = SP // NW
SPB = 96
TPW = N // NW
TCH = 16
NCH = TPW // TCH


def _router_body(x_ref, gw_ref, ri_ref, rf_ref):
    x = x_ref[...]
    gw = gw_ref[...]
    logits = lax.dot_general(gw, x, (((1,), (1,)), ((), ())),
                             preferred_element_type=jnp.float32)
    row = lax.broadcasted_iota(jnp.int32, (E, N), 0)
    big = jnp.int32(E)

    m0 = jnp.max(logits, axis=0, keepdims=True)
    e0 = jnp.min(jnp.where(logits == m0, row, big), axis=0, keepdims=True)
    mask0 = row == e0

    l2 = jnp.where(mask0, -jnp.inf, logits)
    m1 = jnp.max(l2, axis=0, keepdims=True)
    e1 = jnp.min(jnp.where(l2 == m1, row, big), axis=0, keepdims=True)
    mask1 = row == e1

    ex = jnp.exp(logits - m0)
    z = jnp.sum(ex, axis=0, keepdims=True)
    p = ex / z
    v0 = jnp.sum(jnp.where(mask0, p, 0.0), axis=0, keepdims=True)
    v1 = jnp.sum(jnp.where(mask1, p, 0.0), axis=0, keepdims=True)
    vs = v0 + v1
    w0 = v0 / vs
    w1 = v1 / vs

    cnt = (mask0.astype(jnp.bfloat16) + mask1.astype(jnp.bfloat16))
    tr = lax.broadcasted_iota(jnp.int32, (N, N), 0)
    tc = lax.broadcasted_iota(jnp.int32, (N, N), 1)
    ut = (tr < tc).astype(jnp.bfloat16)
    exc = lax.dot_general(cnt, ut, (((1,), (0,)), ((), ())),
                          preferred_element_type=jnp.float32)
    pos0 = jnp.sum(jnp.where(mask0, exc, 0.0), axis=0, keepdims=True)
    pos1 = jnp.sum(jnp.where(mask1, exc, 0.0), axis=0, keepdims=True)
    keep0 = pos0 < float(C)
    keep1 = pos1 < float(C)

    slot0 = e0 * CP + pos0.astype(jnp.int32)
    slot1 = e1 * CP + pos1.astype(jnp.int32)
    scat0 = jnp.where(keep0, slot0, TRASH)
    scat1 = jnp.where(keep1, slot1, TRASH)
    gath0 = jnp.where(keep0, slot0, e0 * CP + C)
    gath1 = jnp.where(keep1, slot1, e1 * CP + C)

    ri_ref[...] = jnp.concatenate([scat0, scat1, gath0, gath1], axis=0)
    rf_ref[...] = jnp.concatenate([jnp.where(keep0, w0, 0.0),
                                   jnp.where(keep1, w1, 0.0)], axis=0)


def _router(x2d, gate_w):
    return pl.pallas_call(
        _router_body,
        out_shape=(jax.ShapeDtypeStruct((4, N), jnp.int32),
                   jax.ShapeDtypeStruct((2, N), jnp.float32)),
    )(x2d, gate_w)


@functools.cache
def _sc_mesh():
    return plsc.VectorSubcoreMesh(core_axis_name="c", subcore_axis_name="s",
                                  num_cores=NC, num_subcores=NS)


@functools.cache
def _dispatch_kernel():
    return pl.kernel(
        _dispatch_body,
        out_type=(jax.ShapeDtypeStruct((SP, D), jnp.float32),
                  jax.ShapeDtypeStruct((SP,), jnp.float32)),
        mesh=_sc_mesh(),
        scratch_types=[
            pltpu.VMEM((N,), jnp.int32),
            pltpu.VMEM((N,), jnp.float32),
            pltpu.VMEM((SPB,), jnp.int32),
            pltpu.VMEM((SPB,), jnp.float32),
            pltpu.VMEM((SPB, D), jnp.float32),
            pltpu.SemaphoreType.DMA,
        ],
        compiler_params=pltpu.CompilerParams(needs_layout_passes=False),
    )


def _dispatch_body(ri_hbm, rf_hbm, x_hbm, eio_hbm, sw_hbm, slots_v, wq_v,
                   loc_t, loc_w, rows_v, sem):
    c = lax.axis_index("c")
    s = lax.axis_index("s")
    wid = s * NC + c
    base = wid * SPW

    def zero(i, carry):
        off = pl.ds(pl.multiple_of(i * 16, 16), 16)
        loc_t[off] = jnp.zeros((16,), jnp.int32)
        loc_w[off] = jnp.zeros((16,), jnp.float32)
        return carry
    lax.fori_loop(0, SPB // 16, zero, 0)
    for k in range(K):
        pltpu.sync_copy(ri_hbm.at[k], slots_v)
        pltpu.sync_copy(rf_hbm.at[k], wq_v)
        def scat(j, carry):
            off = pl.ds(pl.multiple_of(j * 16, 16), 16)
            tok = j * 16 + lax.iota(jnp.int32, 16)
            rel = slots_v[off] - base
            m = (rel >= 0) & (rel < SPW)
            rel = jnp.where(m, rel, 0)
            plsc.store_scatter(loc_t, [rel], tok, mask=m)
            plsc.store_scatter(loc_w, [rel], wq_v[off], mask=m)
            return carry
        lax.fori_loop(0, N // 16, scat, 0)
    pltpu.async_copy(x_hbm.at[loc_t], rows_v, sem).wait()
    pltpu.sync_copy(rows_v.at[pl.ds(0, SPW)], eio_hbm.at[pl.ds(base, SPW)])
    pltpu.sync_copy(loc_w.at[pl.ds(0, SPW)], sw_hbm.at[pl.ds(base, SPW)])


BF = 2048
NFB = F // BF


def _ffn_body(xin_ref, w1_ref, b1_ref, w2_ref, b2_ref, sw_ref, out_ref):
    f = pl.program_id(1)
    xe = xin_ref[0].astype(jnp.bfloat16)
    w1b = w1_ref[0].astype(jnp.bfloat16)
    pre = lax.dot_general(xe, w1b, (((1,), (1,)), ((), ())),
                          preferred_element_type=jnp.float32)
    pre = pre + b1_ref[0, 0][None, :]
    h = (0.5 * pre * (1.0 + lax.erf(pre * 0.7071067811865476))
         ).astype(jnp.bfloat16)
    w2b = w2_ref[0].astype(jnp.bfloat16)
    acc = lax.dot_general(h, w2b, (((1,), (1,)), ((), ())),
                          preferred_element_type=jnp.float32)

    @pl.when(f == 0)
    def _init():
        out_ref[0] = acc

    @pl.when((f != 0) & (f != NFB - 1))
    def _acc():
        out_ref[0] += acc

    @pl.when(f == NFB - 1)
    def _fin():
        out_ref[0] = (out_ref[0] + acc + b2_ref[0, 0][None, :]) * sw_ref[0]


def _ffn(expert_in, w1, b1, w2, b2, slot_w):
    return pl.pallas_call(
        _ffn_body,
        grid=(E, NFB),
        in_specs=[
            pl.BlockSpec((1, CP, D), lambda e, f: (e, 0, 0)),
            pl.BlockSpec((1, BF, D), lambda e, f: (e, f, 0)),
            pl.BlockSpec((1, 1, BF), lambda e, f: (e, 0, f)),
            pl.BlockSpec((1, D, BF), lambda e, f: (e, 0, f)),
            pl.BlockSpec((1, 1, D), lambda e, f: (e, 0, 0)),
            pl.BlockSpec((1, CP, 1), lambda e, f: (e, 0, 0)),
        ],
        out_specs=pl.BlockSpec((1, CP, D), lambda e, f: (e, 0, 0)),
        out_shape=jax.ShapeDtypeStruct((E, CP, D), jnp.float32),
        compiler_params=pltpu.CompilerParams(
            dimension_semantics=("parallel", "arbitrary")),
    )(expert_in.reshape(E, CP, D), w1, b1.reshape(E, 1, F), w2,
      b2.reshape(E, 1, D), slot_w.reshape(E, CP, 1))


@functools.cache
def _combine_kernel():
    return pl.kernel(
        _combine_body,
        out_type=jax.ShapeDtypeStruct((N, D), jnp.float32),
        mesh=_sc_mesh(),
        scratch_types=[
            pltpu.VMEM((TPW,), jnp.int32),
            pltpu.VMEM((TPW,), jnp.int32),
            pltpu.VMEM((2, TCH, D), jnp.float32),
            pltpu.VMEM((2, TCH, D), jnp.float32),
            pltpu.SemaphoreType.DMA,
            pltpu.SemaphoreType.DMA,
            pltpu.SemaphoreType.DMA,
            pltpu.SemaphoreType.DMA,
        ],
        compiler_params=pltpu.CompilerParams(needs_layout_passes=False),
    )


def _combine_body(ri_hbm, eo_hbm, out_hbm, ia_v, ib_v, ba_v, bb_v,
                  sa0, sa1, sb0, sb1):
    c = lax.axis_index("c")
    s = lax.axis_index("s")
    wid = s * NC + c
    tb = wid * TPW
    sa = (sa0, sa1)
    sb = (sb0, sb1)

    pltpu.sync_copy(ri_hbm.at[2, pl.ds(tb, TPW)], ia_v)
    pltpu.sync_copy(ri_hbm.at[3, pl.ds(tb, TPW)], ib_v)

    pend = [None, None]

    def start(ch):
        b = ch & 1
        ia = ia_v.at[pl.ds(ch * TCH, TCH)]
        ib = ib_v.at[pl.ds(ch * TCH, TCH)]
        pend[b] = (pltpu.async_copy(eo_hbm.at[ia], ba_v.at[b], sa[b]),
                   pltpu.async_copy(eo_hbm.at[ib], bb_v.at[b], sb[b]))

    start(0)
    for ch in range(NCH):
        b = ch & 1
        cp_a, cp_b = pend[b]
        cp_a.wait()
        cp_b.wait()
        if ch + 1 < NCH:
            start(ch + 1)

        def addrow(i, carry):
            for j in range(D // 16):
                sl = pl.ds(j * 16, 16)
                ba_v[b, i, sl] = ba_v[b, i, sl] + bb_v[b, i, sl]
            return carry
        lax.fori_loop(0, TCH, addrow, 0)
        pltpu.sync_copy(ba_v.at[b], out_hbm.at[pl.ds(tb + ch * TCH, TCH)])


def kernel(x, gate_w, w1, b1, w2, b2):
    x2d = x.reshape(N, D)
    ri, rf = _router(x2d, gate_w)
    expert_in, slot_w = _dispatch_kernel()(ri, rf, x2d)
    expert_out = _ffn(expert_in, w1, b1, w2, b2, slot_w)
    out2d = _combine_kernel()(ri, expert_out.reshape(SP, D))
    return out2d.reshape(x.shape)


# speedup vs baseline: 1.0057x; 1.0057x over previous
import functools

import jax
import jax.numpy as jnp
from jax import lax
from jax.experimental import pallas as pl
from jax.experimental.pallas import tpu as pltpu
from jax.experimental.pallas import tpu_sc as plsc

D = 1024
F = 4096
E = 8
K = 2
N = 2048
C = 320
CP = 352
SP = E * CP
TRASH = 4096

NC, NS = 2, 16
NW = NC * NS
SPW = SP // NW
SPB = 96
TPW = N // NW
TCH = 16
NCH = TPW // TCH


def _router_body(x_ref, gw_ref, ri_ref, rf_ref):
    x = x_ref[...]
    gw = gw_ref[...]
    logits = lax.dot_general(gw, x, (((1,), (1,)), ((), ())),
                             preferred_element_type=jnp.float32)
    row = lax.broadcasted_iota(jnp.int32, (E, N), 0)
    big = jnp.int32(E)

    m0 = jnp.max(logits, axis=0, keepdims=True)
    e0 = jnp.min(jnp.where(logits == m0, row, big), axis=0, keepdims=True)
    mask0 = row == e0

    l2 = jnp.where(mask0, -jnp.inf, logits)
    m1 = jnp.max(l2, axis=0, keepdims=True)
    e1 = jnp.min(jnp.where(l2 == m1, row, big), axis=0, keepdims=True)
    mask1 = row == e1

    ex = jnp.exp(logits - m0)
    z = jnp.sum(ex, axis=0, keepdims=True)
    p = ex / z
    v0 = jnp.sum(jnp.where(mask0, p, 0.0), axis=0, keepdims=True)
    v1 = jnp.sum(jnp.where(mask1, p, 0.0), axis=0, keepdims=True)
    vs = v0 + v1
    w0 = v0 / vs
    w1 = v1 / vs

    cnt = (mask0.astype(jnp.bfloat16) + mask1.astype(jnp.bfloat16))
    tr = lax.broadcasted_iota(jnp.int32, (N, N), 0)
    tc = lax.broadcasted_iota(jnp.int32, (N, N), 1)
    ut = (tr < tc).astype(jnp.bfloat16)
    exc = lax.dot_general(cnt, ut, (((1,), (0,)), ((), ())),
                          preferred_element_type=jnp.float32)
    pos0 = jnp.sum(jnp.where(mask0, exc, 0.0), axis=0, keepdims=True)
    pos1 = jnp.sum(jnp.where(mask1, exc, 0.0), axis=0, keepdims=True)
    keep0 = pos0 < float(C)
    keep1 = pos1 < float(C)

    slot0 = e0 * CP + pos0.astype(jnp.int32)
    slot1 = e1 * CP + pos1.astype(jnp.int32)
    scat0 = jnp.where(keep0, slot0, TRASH)
    scat1 = jnp.where(keep1, slot1, TRASH)
    gath0 = jnp.where(keep0, slot0, e0 * CP + C)
    gath1 = jnp.where(keep1, slot1, e1 * CP + C)

    ri_ref[...] = jnp.concatenate([scat0, scat1, gath0, gath1], axis=0)
    rf_ref[...] = jnp.concatenate([jnp.where(keep0, w0, 0.0),
                                   jnp.where(keep1, w1, 0.0)], axis=0)


def _router(x2d, gate_w):
    return pl.pallas_call(
        _router_body,
        out_shape=(jax.ShapeDtypeStruct((4, N), jnp.int32),
                   jax.ShapeDtypeStruct((2, N), jnp.float32)),
    )(x2d, gate_w)


@functools.cache
def _sc_mesh():
    return plsc.VectorSubcoreMesh(core_axis_name="c", subcore_axis_name="s",
                                  num_cores=NC, num_subcores=NS)


@functools.cache
def _dispatch_kernel():
    return pl.kernel(
        _dispatch_body,
        out_type=(jax.ShapeDtypeStruct((SP, D), jnp.float32),
                  jax.ShapeDtypeStruct((SP,), jnp.float32)),
        mesh=_sc_mesh(),
        scratch_types=[
            pltpu.VMEM((N,), jnp.int32),
            pltpu.VMEM((N,), jnp.float32),
            pltpu.VMEM((SPB,), jnp.int32),
            pltpu.VMEM((SPB,), jnp.float32),
            pltpu.VMEM((SPB, D), jnp.float32),
            pltpu.SemaphoreType.DMA,
        ],
        compiler_params=pltpu.CompilerParams(needs_layout_passes=False),
    )


def _dispatch_body(ri_hbm, rf_hbm, x_hbm, eio_hbm, sw_hbm, slots_v, wq_v,
                   loc_t, loc_w, rows_v, sem):
    c = lax.axis_index("c")
    s = lax.axis_index("s")
    wid = s * NC + c
    base = wid * SPW

    @plsc.parallel_loop(0, SPB // 16, 1, unroll=2)
    def zero(i):
        off = pl.ds(pl.multiple_of(i * 16, 16), 16)
        loc_t[off] = jnp.zeros((16,), jnp.int32)
        loc_w[off] = jnp.zeros((16,), jnp.float32)
    for k in range(K):
        pltpu.sync_copy(ri_hbm.at[k], slots_v)
        pltpu.sync_copy(rf_hbm.at[k], wq_v)
        @plsc.parallel_loop(0, N // 16, 1, unroll=4)
        def scat(j):
            off = pl.ds(pl.multiple_of(j * 16, 16), 16)
            tok = j * 16 + lax.iota(jnp.int32, 16)
            rel = slots_v[off] - base
            m = (rel >= 0) & (rel < SPW)
            rel = jnp.where(m, rel, 0)
            plsc.store_scatter(loc_t, [rel], tok, mask=m)
            plsc.store_scatter(loc_w, [rel], wq_v[off], mask=m)
    pltpu.async_copy(x_hbm.at[loc_t], rows_v, sem).wait()
    pltpu.sync_copy(rows_v.at[pl.ds(0, SPW)], eio_hbm.at[pl.ds(base, SPW)])
    pltpu.sync_copy(loc_w.at[pl.ds(0, SPW)], sw_hbm.at[pl.ds(base, SPW)])


BF = 2048
NFB = F // BF


def _ffn_body(xin_ref, w1_ref, b1_ref, w2_ref, b2_ref, sw_ref, out_ref):
    f = pl.program_id(1)
    xe = xin_ref[0].astype(jnp.bfloat16)
    w1b = w1_ref[0].astype(jnp.bfloat16)
    pre = lax.dot_general(xe, w1b, (((1,), (1,)), ((), ())),
                          preferred_element_type=jnp.float32)
    pre = pre + b1_ref[0, 0][None, :]
    h = (0.5 * pre * (1.0 + lax.erf(pre * 0.7071067811865476))
         ).astype(jnp.bfloat16)
    w2b = w2_ref[0].astype(jnp.bfloat16)
    acc = lax.dot_general(h, w2b, (((1,), (1,)), ((), ())),
                          preferred_element_type=jnp.float32)

    @pl.when(f == 0)
    def _init():
        out_ref[0] = acc

    @pl.when((f != 0) & (f != NFB - 1))
    def _acc():
        out_ref[0] += acc

    @pl.when(f == NFB - 1)
    def _fin():
        out_ref[0] = (out_ref[0] + acc + b2_ref[0, 0][None, :]) * sw_ref[0]


def _ffn(expert_in, w1, b1, w2, b2, slot_w):
    return pl.pallas_call(
        _ffn_body,
        grid=(E, NFB),
        in_specs=[
            pl.BlockSpec((1, CP, D), lambda e, f: (e, 0, 0)),
            pl.BlockSpec((1, BF, D), lambda e, f: (e, f, 0)),
            pl.BlockSpec((1, 1, BF), lambda e, f: (e, 0, f)),
            pl.BlockSpec((1, D, BF), lambda e, f: (e, 0, f)),
            pl.BlockSpec((1, 1, D), lambda e, f: (e, 0, 0)),
            pl.BlockSpec((1, CP, 1), lambda e, f: (e, 0, 0)),
        ],
        out_specs=pl.BlockSpec((1, CP, D), lambda e, f: (e, 0, 0)),
        out_shape=jax.ShapeDtypeStruct((E, CP, D), jnp.float32),
        compiler_params=pltpu.CompilerParams(
            dimension_semantics=("parallel", "arbitrary")),
    )(expert_in.reshape(E, CP, D), w1, b1.reshape(E, 1, F), w2,
      b2.reshape(E, 1, D), slot_w.reshape(E, CP, 1))


@functools.cache
def _combine_kernel():
    return pl.kernel(
        _combine_body,
        out_type=jax.ShapeDtypeStruct((N, D), jnp.float32),
        mesh=_sc_mesh(),
        scratch_types=[
            pltpu.VMEM((TPW,), jnp.int32),
            pltpu.VMEM((TPW,), jnp.int32),
            pltpu.VMEM((2, TCH, D), jnp.float32),
            pltpu.VMEM((2, TCH, D), jnp.float32),
            pltpu.SemaphoreType.DMA,
            pltpu.SemaphoreType.DMA,
            pltpu.SemaphoreType.DMA,
            pltpu.SemaphoreType.DMA,
        ],
        compiler_params=pltpu.CompilerParams(needs_layout_passes=False),
    )


def _combine_body(ri_hbm, eo_hbm, out_hbm, ia_v, ib_v, ba_v, bb_v,
                  sa0, sa1, sb0, sb1):
    c = lax.axis_index("c")
    s = lax.axis_index("s")
    wid = s * NC + c
    tb = wid * TPW
    sa = (sa0, sa1)
    sb = (sb0, sb1)

    pltpu.sync_copy(ri_hbm.at[2, pl.ds(tb, TPW)], ia_v)
    pltpu.sync_copy(ri_hbm.at[3, pl.ds(tb, TPW)], ib_v)

    pend = [None, None]

    def start(ch):
        b = ch & 1
        ia = ia_v.at[pl.ds(ch * TCH, TCH)]
        ib = ib_v.at[pl.ds(ch * TCH, TCH)]
        pend[b] = (pltpu.async_copy(eo_hbm.at[ia], ba_v.at[b], sa[b]),
                   pltpu.async_copy(eo_hbm.at[ib], bb_v.at[b], sb[b]))

    start(0)
    for ch in range(NCH):
        b = ch & 1
        cp_a, cp_b = pend[b]
        cp_a.wait()
        cp_b.wait()
        if ch + 1 < NCH:
            start(ch + 1)

        @plsc.parallel_loop(0, TCH, 1, unroll=4)
        def addrow(i):
            for j in range(D // 16):
                sl = pl.ds(j * 16, 16)
                ba_v[b, i, sl] = ba_v[b, i, sl] + bb_v[b, i, sl]
        pltpu.sync_copy(ba_v.at[b], out_hbm.at[pl.ds(tb + ch * TCH, TCH)])


def kernel(x, gate_w, w1, b1, w2, b2):
    x2d = x.reshape(N, D)
    ri, rf = _router(x2d, gate_w)
    expert_in, slot_w = _dispatch_kernel()(ri, rf, x2d)
    expert_out = _ffn(expert_in, w1, b1, w2, b2, slot_w)
    out2d = _combine_kernel()(ri, expert_out.reshape(SP, D))
    return out2d.reshape(x.shape)


# speedup vs baseline: 1.0813x; 1.0752x over previous
import functools

import jax
import jax.numpy as jnp
from jax import lax
from jax.experimental import pallas as pl
from jax.experimental.pallas import tpu as pltpu
from jax.experimental.pallas import tpu_sc as plsc

D = 1024
F = 4096
E = 8
K = 2
N = 2048
C = 320
CP = 352
SP = E * CP
TRASH = 4096

NC, NS = 2, 16
NW = NC * NS
SPW = SP // NW
SPB = 96
TPW = N // NW
TCH = 16
NCH = TPW // TCH


def _router_body(x_ref, gw_ref, ri_ref, rf_ref):
    x = x_ref[...]
    gw = gw_ref[...]
    logits = lax.dot_general(gw, x, (((1,), (1,)), ((), ())),
                             preferred_element_type=jnp.float32)
    row = lax.broadcasted_iota(jnp.int32, (E, N), 0)
    big = jnp.int32(E)

    m0 = jnp.max(logits, axis=0, keepdims=True)
    e0 = jnp.min(jnp.where(logits == m0, row, big), axis=0, keepdims=True)
    mask0 = row == e0

    l2 = jnp.where(mask0, -jnp.inf, logits)
    m1 = jnp.max(l2, axis=0, keepdims=True)
    e1 = jnp.min(jnp.where(l2 == m1, row, big), axis=0, keepdims=True)
    mask1 = row == e1

    ex = jnp.exp(logits - m0)
    z = jnp.sum(ex, axis=0, keepdims=True)
    p = ex / z
    v0 = jnp.sum(jnp.where(mask0, p, 0.0), axis=0, keepdims=True)
    v1 = jnp.sum(jnp.where(mask1, p, 0.0), axis=0, keepdims=True)
    vs = v0 + v1
    w0 = v0 / vs
    w1 = v1 / vs

    cnt = (mask0.astype(jnp.bfloat16) + mask1.astype(jnp.bfloat16))
    tr = lax.broadcasted_iota(jnp.int32, (N, N), 0)
    tc = lax.broadcasted_iota(jnp.int32, (N, N), 1)
    ut = (tr < tc).astype(jnp.bfloat16)
    exc = lax.dot_general(cnt, ut, (((1,), (0,)), ((), ())),
                          preferred_element_type=jnp.float32)
    pos0 = jnp.sum(jnp.where(mask0, exc, 0.0), axis=0, keepdims=True)
    pos1 = jnp.sum(jnp.where(mask1, exc, 0.0), axis=0, keepdims=True)
    keep0 = pos0 < float(C)
    keep1 = pos1 < float(C)

    slot0 = e0 * CP + pos0.astype(jnp.int32)
    slot1 = e1 * CP + pos1.astype(jnp.int32)
    scat0 = jnp.where(keep0, slot0, TRASH)
    scat1 = jnp.where(keep1, slot1, TRASH)
    gath0 = jnp.where(keep0, slot0, e0 * CP + C)
    gath1 = jnp.where(keep1, slot1, e1 * CP + C)

    ri_ref[...] = jnp.concatenate([scat0, scat1, gath0, gath1], axis=0)
    rf_ref[...] = jnp.concatenate([jnp.where(keep0, w0, 0.0),
                                   jnp.where(keep1, w1, 0.0)], axis=0)


def _router(x2d, gate_w):
    return pl.pallas_call(
        _router_body,
        out_shape=(jax.ShapeDtypeStruct((4, N), jnp.int32),
                   jax.ShapeDtypeStruct((2, N), jnp.float32)),
    )(x2d, gate_w)


@functools.cache
def _sc_mesh():
    return plsc.VectorSubcoreMesh(core_axis_name="c", subcore_axis_name="s",
                                  num_cores=NC, num_subcores=NS)


@functools.cache
def _dispatch_kernel():
    return pl.kernel(
        _dispatch_body,
        out_type=(jax.ShapeDtypeStruct((SP, D), jnp.float32),
                  jax.ShapeDtypeStruct((SP,), jnp.float32)),
        mesh=_sc_mesh(),
        scratch_types=[
            pltpu.VMEM((N,), jnp.int32),
            pltpu.VMEM((N,), jnp.float32),
            pltpu.VMEM((SP,), jnp.int32),
            pltpu.VMEM((SP,), jnp.float32),
            pltpu.VMEM_SHARED((SP,), jnp.int32),
            pltpu.VMEM((SPW,), jnp.int32),
            pltpu.VMEM((SPW, D), jnp.float32),
            pltpu.SemaphoreType.DMA,
        ],
        compiler_params=pltpu.CompilerParams(needs_layout_passes=False),
    )


def _dispatch_body(ri_hbm, rf_hbm, x_hbm, eio_hbm, sw_hbm, slots_v, wq_v,
                   s2t_v, s2w_v, s2t_sh, idx_v, rows_v, sem):
    c = lax.axis_index("c")
    s = lax.axis_index("s")
    wid = s * NC + c

    @pl.when(s == 0)
    def _build_t():
        @plsc.parallel_loop(0, SP // 16, 1, unroll=2)
        def zero(i):
            s2t_v[pl.ds(pl.multiple_of(i * 16, 16), 16)] = (
                jnp.zeros((16,), jnp.int32))
        for k in range(K):
            pltpu.sync_copy(ri_hbm.at[k], slots_v)
            @plsc.parallel_loop(0, N // 16, 1, unroll=4)
            def scat(j):
                off = pl.ds(pl.multiple_of(j * 16, 16), 16)
                tok = j * 16 + lax.iota(jnp.int32, 16)
                sl = slots_v[off]
                plsc.store_scatter(s2t_v, [sl], tok, mask=sl < SP)
        pltpu.sync_copy(s2t_v, s2t_sh)

    @pl.when((s == 1) & (c == 0))
    def _build_w():
        @plsc.parallel_loop(0, SP // 16, 1, unroll=2)
        def zero(i):
            s2w_v[pl.ds(pl.multiple_of(i * 16, 16), 16)] = (
                jnp.zeros((16,), jnp.float32))
        for k in range(K):
            pltpu.sync_copy(ri_hbm.at[k], slots_v)
            pltpu.sync_copy(rf_hbm.at[k], wq_v)
            @plsc.parallel_loop(0, N // 16, 1, unroll=4)
            def scat(j):
                off = pl.ds(pl.multiple_of(j * 16, 16), 16)
                sl = slots_v[off]
                plsc.store_scatter(s2w_v, [sl], wq_v[off], mask=sl < SP)
        pltpu.sync_copy(s2w_v, sw_hbm)

    plsc.subcore_barrier()
    base = wid * SPW
    pltpu.sync_copy(s2t_sh.at[pl.ds(base, SPW)], idx_v)
    pltpu.async_copy(x_hbm.at[idx_v], rows_v, sem).wait()
    pltpu.sync_copy(rows_v, eio_hbm.at[pl.ds(base, SPW)])


BF = 2048
NFB = F // BF


def _ffn_body(xin_ref, w1_ref, b1_ref, w2_ref, b2_ref, sw_ref, out_ref):
    f = pl.program_id(1)
    xe = xin_ref[0].astype(jnp.bfloat16)
    w1b = w1_ref[0].astype(jnp.bfloat16)
    pre = lax.dot_general(xe, w1b, (((1,), (1,)), ((), ())),
                          preferred_element_type=jnp.float32)
    pre = pre + b1_ref[0, 0][None, :]
    h = (0.5 * pre * (1.0 + lax.erf(pre * 0.7071067811865476))
         ).astype(jnp.bfloat16)
    w2b = w2_ref[0].astype(jnp.bfloat16)
    acc = lax.dot_general(h, w2b, (((1,), (1,)), ((), ())),
                          preferred_element_type=jnp.float32)

    @pl.when(f == 0)
    def _init():
        out_ref[0] = acc

    @pl.when((f != 0) & (f != NFB - 1))
    def _acc():
        out_ref[0] += acc

    @pl.when(f == NFB - 1)
    def _fin():
        out_ref[0] = (out_ref[0] + acc + b2_ref[0, 0][None, :]) * sw_ref[0]


def _ffn(expert_in, w1, b1, w2, b2, slot_w):
    return pl.pallas_call(
        _ffn_body,
        grid=(E, NFB),
        in_specs=[
            pl.BlockSpec((1, CP, D), lambda e, f: (e, 0, 0)),
            pl.BlockSpec((1, BF, D), lambda e, f: (e, f, 0)),
            pl.BlockSpec((1, 1, BF), lambda e, f: (e, 0, f)),
            pl.BlockSpec((1, D, BF), lambda e, f: (e, 0, f)),
            pl.BlockSpec((1, 1, D), lambda e, f: (e, 0, 0)),
            pl.BlockSpec((1, CP, 1), lambda e, f: (e, 0, 0)),
        ],
        out_specs=pl.BlockSpec((1, CP, D), lambda e, f: (e, 0, 0)),
        out_shape=jax.ShapeDtypeStruct((E, CP, D), jnp.float32),
        compiler_params=pltpu.CompilerParams(
            dimension_semantics=("parallel", "arbitrary")),
    )(expert_in.reshape(E, CP, D), w1, b1.reshape(E, 1, F), w2,
      b2.reshape(E, 1, D), slot_w.reshape(E, CP, 1))


@functools.cache
def _combine_kernel():
    return pl.kernel(
        _combine_body,
        out_type=jax.ShapeDtypeStruct((N, D), jnp.float32),
        mesh=_sc_mesh(),
        scratch_types=[
            pltpu.VMEM((TPW,), jnp.int32),
            pltpu.VMEM((TPW,), jnp.int32),
            pltpu.VMEM((2, TCH, D), jnp.float32),
            pltpu.VMEM((2, TCH, D), jnp.float32),
            pltpu.SemaphoreType.DMA,
            pltpu.SemaphoreType.DMA,
            pltpu.SemaphoreType.DMA,
            pltpu.SemaphoreType.DMA,
        ],
        compiler_params=pltpu.CompilerParams(needs_layout_passes=False),
    )


def _combine_body(ri_hbm, eo_hbm, out_hbm, ia_v, ib_v, ba_v, bb_v,
                  sa0, sa1, sb0, sb1):
    c = lax.axis_index("c")
    s = lax.axis_index("s")
    wid = s * NC + c
    tb = wid * TPW
    sa = (sa0, sa1)
    sb = (sb0, sb1)

    pltpu.sync_copy(ri_hbm.at[2, pl.ds(tb, TPW)], ia_v)
    pltpu.sync_copy(ri_hbm.at[3, pl.ds(tb, TPW)], ib_v)

    pend = [None, None]

    def start(ch):
        b = ch & 1
        ia = ia_v.at[pl.ds(ch * TCH, TCH)]
        ib = ib_v.at[pl.ds(ch * TCH, TCH)]
        pend[b] = (pltpu.async_copy(eo_hbm.at[ia], ba_v.at[b], sa[b]),
                   pltpu.async_copy(eo_hbm.at[ib], bb_v.at[b], sb[b]))

    start(0)
    for ch in range(NCH):
        b = ch & 1
        cp_a, cp_b = pend[b]
        cp_a.wait()
        cp_b.wait()
        if ch + 1 < NCH:
            start(ch + 1)

        @plsc.parallel_loop(0, TCH, 1, unroll=4)
        def addrow(i):
            for j in range(D // 16):
                sl = pl.ds(j * 16, 16)
                ba_v[b, i, sl] = ba_v[b, i, sl] + bb_v[b, i, sl]
        pltpu.sync_copy(ba_v.at[b], out_hbm.at[pl.ds(tb + ch * TCH, TCH)])


def kernel(x, gate_w, w1, b1, w2, b2):
    x2d = x.reshape(N, D)
    ri, rf = _router(x2d, gate_w)
    expert_in, slot_w = _dispatch_kernel()(ri, rf, x2d)
    expert_out = _ffn(expert_in, w1, b1, w2, b2, slot_w)
    out2d = _combine_kernel()(ri, expert_out.reshape(SP, D))
    return out2d.reshape(x.shape)


# speedup vs baseline: 1.0959x; 1.0135x over previous
import functools

import jax
import jax.numpy as jnp
from jax import lax
from jax.experimental import pallas as pl
from jax.experimental.pallas import tpu as pltpu
from jax.experimental.pallas import tpu_sc as plsc

D = 1024
F = 4096
E = 8
K = 2
N = 2048
C = 320
CP = 352
SP = E * CP
TRASH = 4096

NC, NS = 2, 16
NW = NC * NS
SPW = SP // NW
SPB = 96
TPW = N // NW
TCH = 16
NCH = TPW // TCH


def _router_body(x_ref, gw_ref, ri_ref, rf_ref):
    x = x_ref[...]
    gw = gw_ref[...]
    logits = lax.dot_general(gw, x, (((1,), (1,)), ((), ())),
                             preferred_element_type=jnp.float32)
    row = lax.broadcasted_iota(jnp.int32, (E, N), 0)
    big = jnp.int32(E)

    m0 = jnp.max(logits, axis=0, keepdims=True)
    e0 = jnp.min(jnp.where(logits == m0, row, big), axis=0, keepdims=True)
    mask0 = row == e0

    l2 = jnp.where(mask0, -jnp.inf, logits)
    m1 = jnp.max(l2, axis=0, keepdims=True)
    e1 = jnp.min(jnp.where(l2 == m1, row, big), axis=0, keepdims=True)
    mask1 = row == e1

    ex = jnp.exp(logits - m0)
    z = jnp.sum(ex, axis=0, keepdims=True)
    p = ex / z
    v0 = jnp.sum(jnp.where(mask0, p, 0.0), axis=0, keepdims=True)
    v1 = jnp.sum(jnp.where(mask1, p, 0.0), axis=0, keepdims=True)
    vs = v0 + v1
    w0 = v0 / vs
    w1 = v1 / vs

    cnt = (mask0.astype(jnp.bfloat16) + mask1.astype(jnp.bfloat16))
    tr = lax.broadcasted_iota(jnp.int32, (N, N), 0)
    tc = lax.broadcasted_iota(jnp.int32, (N, N), 1)
    ut = (tr < tc).astype(jnp.bfloat16)
    exc = lax.dot_general(cnt, ut, (((1,), (0,)), ((), ())),
                          preferred_element_type=jnp.float32)
    pos0 = jnp.sum(jnp.where(mask0, exc, 0.0), axis=0, keepdims=True)
    pos1 = jnp.sum(jnp.where(mask1, exc, 0.0), axis=0, keepdims=True)
    keep0 = pos0 < float(C)
    keep1 = pos1 < float(C)

    slot0 = e0 * CP + pos0.astype(jnp.int32)
    slot1 = e1 * CP + pos1.astype(jnp.int32)
    scat0 = jnp.where(keep0, slot0, TRASH)
    scat1 = jnp.where(keep1, slot1, TRASH)
    gath0 = jnp.where(keep0, slot0, e0 * CP + C)
    gath1 = jnp.where(keep1, slot1, e1 * CP + C)

    ri_ref[...] = jnp.concatenate([scat0, scat1, gath0, gath1], axis=0)
    rf_ref[...] = jnp.concatenate([jnp.where(keep0, w0, 0.0),
                                   jnp.where(keep1, w1, 0.0)], axis=0)


def _router(x2d, gate_w):
    return pl.pallas_call(
        _router_body,
        out_shape=(jax.ShapeDtypeStruct((4, N), jnp.int32),
                   jax.ShapeDtypeStruct((2, N), jnp.float32)),
    )(x2d, gate_w)


@functools.cache
def _sc_mesh():
    return plsc.VectorSubcoreMesh(core_axis_name="c", subcore_axis_name="s",
                                  num_cores=NC, num_subcores=NS)


@functools.cache
def _dispatch_kernel():
    return pl.kernel(
        _dispatch_body,
        out_type=(jax.ShapeDtypeStruct((SP, D), jnp.float32),
                  jax.ShapeDtypeStruct((SP,), jnp.float32)),
        mesh=_sc_mesh(),
        scratch_types=[
            pltpu.VMEM((N,), jnp.int32),
            pltpu.VMEM((N,), jnp.float32),
            pltpu.VMEM((SP,), jnp.int32),
            pltpu.VMEM((SP,), jnp.float32),
            pltpu.VMEM_SHARED((SP,), jnp.int32),
            pltpu.VMEM((SPW,), jnp.int32),
            pltpu.VMEM((SPW, D), jnp.float32),
            pltpu.SemaphoreType.DMA,
        ],
        compiler_params=pltpu.CompilerParams(needs_layout_passes=False),
    )


def _dispatch_body(ri_hbm, rf_hbm, x_hbm, eio_hbm, sw_hbm, slots_v, wq_v,
                   s2t_v, s2w_v, s2t_sh, idx_v, rows_v, sem):
    c = lax.axis_index("c")
    s = lax.axis_index("s")
    wid = s * NC + c

    @pl.when(s == 0)
    def _build_t():
        @plsc.parallel_loop(0, SP // 16, 1, unroll=2)
        def zero(i):
            s2t_v[pl.ds(pl.multiple_of(i * 16, 16), 16)] = (
                jnp.zeros((16,), jnp.int32))
        for k in range(K):
            pltpu.sync_copy(ri_hbm.at[k], slots_v)
            @plsc.parallel_loop(0, N // 16, 1, unroll=4)
            def scat(j):
                off = pl.ds(pl.multiple_of(j * 16, 16), 16)
                tok = j * 16 + lax.iota(jnp.int32, 16)
                sl = slots_v[off]
                plsc.store_scatter(s2t_v, [sl], tok, mask=sl < SP)
        pltpu.sync_copy(s2t_v, s2t_sh)

    @pl.when((s == 1) & (c == 0))
    def _build_w():
        @plsc.parallel_loop(0, SP // 16, 1, unroll=2)
        def zero(i):
            s2w_v[pl.ds(pl.multiple_of(i * 16, 16), 16)] = (
                jnp.zeros((16,), jnp.float32))
        for k in range(K):
            pltpu.sync_copy(ri_hbm.at[k], slots_v)
            pltpu.sync_copy(rf_hbm.at[k], wq_v)
            @plsc.parallel_loop(0, N // 16, 1, unroll=4)
            def scat(j):
                off = pl.ds(pl.multiple_of(j * 16, 16), 16)
                sl = slots_v[off]
                plsc.store_scatter(s2w_v, [sl], wq_v[off], mask=sl < SP)
        pltpu.sync_copy(s2w_v, sw_hbm)

    plsc.subcore_barrier()
    base = wid * SPW
    pltpu.sync_copy(s2t_sh.at[pl.ds(base, SPW)], idx_v)
    pltpu.async_copy(x_hbm.at[idx_v], rows_v, sem).wait()
    pltpu.sync_copy(rows_v, eio_hbm.at[pl.ds(base, SPW)])


BF = 2048
NFB = F // BF


def _ffn_body(xin_ref, w1_ref, b1_ref, w2_ref, b2_ref, sw_ref, out_ref):
    f = pl.program_id(1)
    xe = xin_ref[0].astype(jnp.bfloat16)
    w1b = w1_ref[0].astype(jnp.bfloat16)
    pre = lax.dot_general(xe, w1b, (((1,), (1,)), ((), ())),
                          preferred_element_type=jnp.float32)
    pre = pre + b1_ref[0, 0][None, :]
    h = (0.5 * pre * (1.0 + lax.erf(pre * 0.7071067811865476))
         ).astype(jnp.bfloat16)
    w2b = w2_ref[0].astype(jnp.bfloat16)
    acc = lax.dot_general(h, w2b, (((1,), (1,)), ((), ())),
                          preferred_element_type=jnp.float32)

    @pl.when(f == 0)
    def _init():
        out_ref[0] = acc

    @pl.when((f != 0) & (f != NFB - 1))
    def _acc():
        out_ref[0] += acc

    @pl.when(f == NFB - 1)
    def _fin():
        out_ref[0] = (out_ref[0] + acc + b2_ref[0, 0][None, :]) * sw_ref[0]


def _ffn(expert_in, w1, b1, w2, b2, slot_w):
    return pl.pallas_call(
        _ffn_body,
        grid=(E, NFB),
        in_specs=[
            pl.BlockSpec((1, CP, D), lambda e, f: (e, 0, 0)),
            pl.BlockSpec((1, BF, D), lambda e, f: (e, f, 0)),
            pl.BlockSpec((1, 1, BF), lambda e, f: (e, 0, f)),
            pl.BlockSpec((1, D, BF), lambda e, f: (e, 0, f)),
            pl.BlockSpec((1, 1, D), lambda e, f: (e, 0, 0)),
            pl.BlockSpec((1, CP, 1), lambda e, f: (e, 0, 0)),
        ],
        out_specs=pl.BlockSpec((1, CP, D), lambda e, f: (e, 0, 0)),
        out_shape=jax.ShapeDtypeStruct((E, CP, D), jnp.float32),
        compiler_params=pltpu.CompilerParams(
            dimension_semantics=("parallel", "arbitrary")),
    )(expert_in.reshape(E, CP, D), w1, b1.reshape(E, 1, F), w2,
      b2.reshape(E, 1, D), slot_w.reshape(E, CP, 1))


@functools.cache
def _combine_kernel():
    return pl.kernel(
        _combine_body,
        out_type=jax.ShapeDtypeStruct((N, D), jnp.float32),
        mesh=_sc_mesh(),
        scratch_types=[
            pltpu.VMEM((NCH, TCH), jnp.int32),
            pltpu.VMEM((NCH, TCH), jnp.int32),
            pltpu.VMEM((2, TCH, D), jnp.float32),
            pltpu.VMEM((2, TCH, D), jnp.float32),
            pltpu.SemaphoreType.DMA,
            pltpu.SemaphoreType.DMA,
            pltpu.SemaphoreType.DMA,
            pltpu.SemaphoreType.DMA,
        ],
        compiler_params=pltpu.CompilerParams(needs_layout_passes=False),
    )


def _combine_body(ga_hbm, gb_hbm, eo_hbm, out_hbm, ia_v, ib_v, ba_v, bb_v,
                  sa0, sa1, sb0, sb1):
    c = lax.axis_index("c")
    s = lax.axis_index("s")
    wid = s * NC + c
    tb = wid * TPW
    sa = (sa0, sa1)
    sb = (sb0, sb1)

    pltpu.sync_copy(ga_hbm.at[wid], ia_v)
    pltpu.sync_copy(gb_hbm.at[wid], ib_v)

    pend = [None, None]

    def start(ch):
        b = ch & 1
        pend[b] = (pltpu.async_copy(eo_hbm.at[ia_v.at[ch]], ba_v.at[b], sa[b]),
                   pltpu.async_copy(eo_hbm.at[ib_v.at[ch]], bb_v.at[b], sb[b]))

    start(0)
    for ch in range(NCH):
        b = ch & 1
        cp_a, cp_b = pend[b]
        cp_a.wait()
        cp_b.wait()
        if ch + 1 < NCH:
            start(ch + 1)

        @plsc.parallel_loop(0, TCH, 1, unroll=4)
        def addrow(i):
            for j in range(D // 16):
                sl = pl.ds(j * 16, 16)
                ba_v[b, i, sl] = ba_v[b, i, sl] + bb_v[b, i, sl]
        pltpu.sync_copy(ba_v.at[b], out_hbm.at[pl.ds(tb + ch * TCH, TCH)])


def kernel(x, gate_w, w1, b1, w2, b2):
    x2d = x.reshape(N, D)
    ri, rf = _router(x2d, gate_w)
    expert_in, slot_w = _dispatch_kernel()(ri, rf, x2d)
    expert_out = _ffn(expert_in, w1, b1, w2, b2, slot_w)
    ga = ri[2].reshape(NW, NCH, TCH)
    gb = ri[3].reshape(NW, NCH, TCH)
    out2d = _combine_kernel()(ga, gb, expert_out.reshape(SP, D))
    return out2d.reshape(x.shape)
